# baseline (speedup 1.0000x reference)
import functools

import jax
import jax.numpy as jnp
from jax import lax
from jax.experimental import pallas as pl
from jax.experimental.pallas import tpu as pltpu
from jax.experimental.pallas import tpu_sc as plsc

VOCAB = 1000000
EMB = 32
B = 4096
L = 200

NC = 2
NS = 16
NW = NC * NS
B_PER_W = B // NW
NE8 = EMB // 8
NBUF = 2
VC = VOCAB // 4

_mesh = plsc.VectorSubcoreMesh(core_axis_name="c", subcore_axis_name="s")


@functools.partial(
    pl.kernel,
    out_type=jax.ShapeDtypeStruct((L, NE8, NW, 8, B_PER_W), jnp.float32),
    mesh=_mesh,
    scratch_types=[
        pltpu.VMEM((B_PER_W, L), jnp.int32),
        pltpu.VMEM((L, B_PER_W), jnp.int32),
        pltpu.VMEM((L, B_PER_W), jnp.int32),
        pltpu.VMEM((NBUF, B_PER_W, EMB * 4), jnp.float32),
        pltpu.VMEM((NBUF, EMB, B_PER_W), jnp.float32),
        pltpu.SemaphoreType.DMA((NBUF,)),
        pltpu.SemaphoreType.DMA((NBUF,)),
    ],
    compiler_params=pltpu.CompilerParams(
        use_tc_tiling_on_sc=False,
        needs_layout_passes=False,
        disable_bounds_checks=True,
    ),
)
def _embed_sc(idx_hbm, table_hbm, out_hbm, idx_raw, idx_t, qof_t, rows_v,
              tr_v, gsem, wsem):
    wid = lax.axis_index("s") * NC + lax.axis_index("c")

    pltpu.sync_copy(idx_hbm.at[wid], idx_raw)

    lanes = lax.iota(jnp.int32, 16)
    diag = [lax.rem(lanes + k, 16) for k in range(16)]

    def idx_tbody(cb, _):
        c0 = cb * 16
        for rb in range(8):
            rvec = rb * 16 + lanes
            for k in range(16):
                cvec = c0 + diag[k]
                v = plsc.load_gather(idx_raw, [rvec, cvec])
                plsc.store_scatter(idx_t, [cvec, rvec],
                                   lax.shift_right_logical(v, 2))
                plsc.store_scatter(qof_t, [cvec, rvec],
                                   lax.shift_left(v & 3, 5))
        return 0

    lax.fori_loop(0, L // 16, idx_tbody, 0)

    for rb in range(8):
        rvec = rb * 16 + lanes
        for k in range(8):
            cvec = 192 + lax.rem(lanes + k, 8)
            v = plsc.load_gather(idx_raw, [rvec, cvec])
            plsc.store_scatter(idx_t, [cvec, rvec],
                               lax.shift_right_logical(v, 2))
            plsc.store_scatter(qof_t, [cvec, rvec],
                               lax.shift_left(v & 3, 5))

    def fire_gather(l, b):
        pltpu.async_copy(table_hbm.at[idx_t.at[l]], rows_v.at[b], gsem.at[b])

    def drain_gather(b):
        pltpu.make_async_copy(
            table_hbm.at[pl.ds(0, B_PER_W)], rows_v.at[b], gsem.at[b]
        ).wait()

    def drain_wb(b):
        for te in range(NE8):
            pltpu.make_async_copy(
                out_hbm.at[0, te, 0], tr_v.at[b, pl.ds(te * 8, 8)], wsem.at[b]
            ).wait()

    for b in range(NBUF):
        fire_gather(b, b)

    def body(i, _):
        for b in range(NBUF):
            l = i * NBUF + b
            drain_gather(b)

            @pl.when(l >= NBUF)
            def _():
                drain_wb(b)

            rows_ref = rows_v.at[b]
            tr_ref = tr_v.at[b]

            def trans_body(rb, _):
                rvec = rb * 16 + lanes
                qvec = qof_t[l, pl.ds(rb * 16, 16)]
                for cb in range(2):
                    for k in range(16):
                        cvec = qvec + (cb * 16 + diag[k])
                        v = plsc.load_gather(rows_ref, [rvec, cvec])
                        plsc.store_scatter(
                            tr_ref, [cb * 16 + diag[k], rvec], v
                        )
                return 0

            lax.fori_loop(0, 8, trans_body, 0)

            for te in range(NE8):
                pltpu.async_copy(
                    tr_v.at[b, pl.ds(te * 8, 8)], out_hbm.at[l, te, wid],
                    wsem.at[b],
                )

            @pl.when(l + NBUF < L)
            def _():
                fire_gather(l + NBUF, b)

        return 0

    lax.fori_loop(0, L // NBUF, body, 0)

    for b in range(NBUF):
        drain_wb(b)


def kernel(x, single, emb_weight):
    idx = x.reshape(NW, B_PER_W, L).astype(jnp.int32)
    p = _embed_sc(idx, emb_weight.reshape(VC, EMB * 4))
    return p.transpose(2, 4, 0, 1, 3).reshape(B, L, EMB)


# speedup vs baseline: 1.0821x; 1.0821x over previous
import functools

import jax
import jax.numpy as jnp
from jax import lax
from jax.experimental import pallas as pl
from jax.experimental.pallas import tpu as pltpu
from jax.experimental.pallas import tpu_sc as plsc

VOCAB = 1000000
EMB = 32
B = 4096
L = 200

NC = 2
NS = 16
NW = NC * NS
B_PER_W = B // NW
NE8 = EMB // 8
NBUF = 4

_mesh = plsc.VectorSubcoreMesh(core_axis_name="c", subcore_axis_name="s")


@functools.partial(
    pl.kernel,
    out_type=jax.ShapeDtypeStruct((L, NE8, NW, 8, B_PER_W), jnp.float32),
    mesh=_mesh,
    scratch_types=[
        pltpu.VMEM((B_PER_W, L), jnp.int32),
        pltpu.VMEM((L, B_PER_W), jnp.int32),
        pltpu.VMEM((NBUF, B_PER_W, EMB), jnp.float32),
        pltpu.VMEM((NBUF, EMB, B_PER_W), jnp.float32),
        pltpu.SemaphoreType.DMA((NBUF,)),
        pltpu.SemaphoreType.DMA((NBUF,)),
    ],
    compiler_params=pltpu.CompilerParams(
        use_tc_tiling_on_sc=False,
        needs_layout_passes=False,
        disable_bounds_checks=True,
    ),
)
def _embed_sc(idx_hbm, table_hbm, out_hbm, idx_raw, idx_t, rows_v, tr_v,
              gsem, wsem):
    wid = lax.axis_index("s") * NC + lax.axis_index("c")

    pltpu.sync_copy(idx_hbm.at[wid], idx_raw)

    lanes = lax.iota(jnp.int32, 16)
    diag = [lax.rem(lanes + k, 16) for k in range(16)]

    def idx_tbody(cb, _):
        c0 = cb * 16
        for rb in range(8):
            rvec = rb * 16 + lanes
            for k in range(16):
                cvec = c0 + diag[k]
                v = plsc.load_gather(idx_raw, [rvec, cvec])
                plsc.store_scatter(idx_t, [cvec, rvec], v)
        return 0

    lax.fori_loop(0, L // 16, idx_tbody, 0)

    for rb in range(8):
        rvec = rb * 16 + lanes
        for k in range(8):
            cvec = 192 + lax.rem(lanes + k, 8)
            v = plsc.load_gather(idx_raw, [rvec, cvec])
            plsc.store_scatter(idx_t, [cvec, rvec], v)

    def fire_gather(l, b):
        pltpu.async_copy(table_hbm.at[idx_t.at[l]], rows_v.at[b], gsem.at[b])

    def drain_gather(b):
        pltpu.make_async_copy(
            table_hbm.at[pl.ds(0, B_PER_W)], rows_v.at[b], gsem.at[b]
        ).wait()

    def drain_wb(b):
        for te in range(NE8):
            pltpu.make_async_copy(
                out_hbm.at[0, te, 0], tr_v.at[b, pl.ds(te * 8, 8)], wsem.at[b]
            ).wait()

    for b in range(NBUF):
        fire_gather(b, b)

    def body(i, _):
        for b in range(NBUF):
            l = i * NBUF + b
            drain_gather(b)

            @pl.when(l >= NBUF)
            def _():
                drain_wb(b)

            rows_ref = rows_v.at[b]
            tr_ref = tr_v.at[b]

            def trans_body(rb, _):
                rvec = rb * 16 + lanes
                for cb in range(2):
                    for k in range(16):
                        cvec = cb * 16 + diag[k]
                        v = plsc.load_gather(rows_ref, [rvec, cvec])
                        plsc.store_scatter(tr_ref, [cvec, rvec], v)
                return 0

            lax.fori_loop(0, 8, trans_body, 0)

            for te in range(NE8):
                pltpu.async_copy(
                    tr_v.at[b, pl.ds(te * 8, 8)], out_hbm.at[l, te, wid],
                    wsem.at[b],
                )

            @pl.when(l + NBUF < L)
            def _():
                fire_gather(l + NBUF, b)

        return 0

    lax.fori_loop(0, L // NBUF, body, 0)

    for b in range(NBUF):
        drain_wb(b)


def kernel(x, single, emb_weight):
    idx = x.reshape(NW, B_PER_W, L).astype(jnp.int32)
    p = _embed_sc(idx, emb_weight)
    return p.transpose(2, 4, 0, 1, 3).reshape(B, L, EMB)
